# baseline (speedup 1.0000x reference)
import jax
import jax.numpy as jnp
from jax import lax
from jax.experimental import pallas as pl
from jax.experimental.pallas import tpu as pltpu
from jax.experimental.pallas import tpu_sc as plsc

D_OUT = 4096
D_IN = 4096
BLOCK = 64
N_BLOCKS = D_IN // BLOCK
LANES = 16
GROUPS = N_BLOCKS // LANES

NC = 2
NS = 16
NW = NC * NS
ROWS_PER_W = D_OUT // NW
CHUNK_ROWS = 4
N_CHUNKS = ROWS_PER_W // CHUNK_ROWS
NBUF = 2
N_PAIRS = N_CHUNKS // NBUF


def _compute_chunk(src, dst, sbuf):
    def row_group(i, carry):
        r = i // GROUPS
        g = i % GROUPS
        sv = sbuf[r, pl.ds(g * LANES, LANES)]
        ssafe = jnp.where(sv == 0.0, 1.0, sv)
        r75v = 7.5 / ssafe
        mv = sv * (2.0 / 15.0)
        dnums = lax.GatherDimensionNumbers(
            offset_dims=(), collapsed_slice_dims=(0,), start_index_map=(0,))
        for k in range(LANES):
            idx = jnp.full((LANES, 1), k, jnp.int32)
            r75s = lax.gather(r75v, idx, dnums, (1,),
                              mode=lax.GatherScatterMode.PROMISE_IN_BOUNDS)
            ms = lax.gather(mv, idx, dnums, (1,),
                            mode=lax.GatherScatterMode.PROMISE_IN_BOUNDS)
            colbase = g * (LANES * BLOCK) + k * BLOCK
            for v in range(BLOCK // LANES):
                col = colbase + v * LANES
                x = src[r, pl.ds(col, LANES)]
                u = x * r75s + 8.0
                cnt = u.astype(jnp.int32).astype(jnp.float32)
                dst[r, pl.ds(col, LANES)] = (cnt - 7.5) * ms
        return carry

    lax.fori_loop(0, CHUNK_ROWS * GROUPS, row_group, 0)


def _sc_body(master_hbm, scale_hbm, out_hbm,
             bin0, bin1, bout0, bout1, sb0, sb1,
             isem0, isem1, osem0, osem1):
    bufs_in = (bin0, bin1)
    bufs_out = (bout0, bout1)
    sbufs = (sb0, sb1)
    isems = (isem0, isem1)
    osems = (osem0, osem1)

    wid = lax.axis_index("s") * NC + lax.axis_index("c")
    base = wid * ROWS_PER_W

    def prefetch(c, b):
        r0 = base + c * CHUNK_ROWS
        pltpu.async_copy(master_hbm.at[pl.ds(r0, CHUNK_ROWS)],
                         bufs_in[b], isems[b])
        pltpu.async_copy(scale_hbm.at[pl.ds(r0, CHUNK_ROWS)],
                         sbufs[b], isems[b])

    def wait_in(b):
        pltpu.make_async_copy(master_hbm.at[pl.ds(0, CHUNK_ROWS)],
                              bufs_in[b], isems[b]).wait()
        pltpu.make_async_copy(scale_hbm.at[pl.ds(0, CHUNK_ROWS)],
                              sbufs[b], isems[b]).wait()

    def wait_out(b):
        pltpu.make_async_copy(bufs_out[b],
                              out_hbm.at[pl.ds(0, CHUNK_ROWS)],
                              osems[b]).wait()

    for b in range(NBUF):
        prefetch(b, b)

    def pair_body(p, carry):
        for b in range(NBUF):
            c = p * NBUF + b
            wait_in(b)

            @pl.when(p > 0)
            def _():
                wait_out(b)

            _compute_chunk(bufs_in[b], bufs_out[b], sbufs[b])

            r0 = base + c * CHUNK_ROWS
            pltpu.async_copy(bufs_out[b],
                             out_hbm.at[pl.ds(r0, CHUNK_ROWS)], osems[b])

            @pl.when(p < N_PAIRS - 1)
            def _():
                prefetch(c + NBUF, b)
        return carry

    lax.fori_loop(0, N_PAIRS, pair_body, 0)

    for b in range(NBUF):
        wait_out(b)


def kernel(master, scale, centroids):
    del centroids
    mesh = plsc.VectorSubcoreMesh(core_axis_name="c", subcore_axis_name="s")
    k = pl.kernel(
        _sc_body,
        mesh=mesh,
        out_type=jax.ShapeDtypeStruct((D_OUT, D_IN), jnp.float32),
        scratch_types=[
            pltpu.VMEM((CHUNK_ROWS, D_IN), jnp.float32),
            pltpu.VMEM((CHUNK_ROWS, D_IN), jnp.float32),
            pltpu.VMEM((CHUNK_ROWS, D_IN), jnp.float32),
            pltpu.VMEM((CHUNK_ROWS, D_IN), jnp.float32),
            pltpu.VMEM((CHUNK_ROWS, N_BLOCKS), jnp.float32),
            pltpu.VMEM((CHUNK_ROWS, N_BLOCKS), jnp.float32),
            pltpu.SemaphoreType.DMA,
            pltpu.SemaphoreType.DMA,
            pltpu.SemaphoreType.DMA,
            pltpu.SemaphoreType.DMA,
        ],
    )
    return k(master, scale)


# speedup vs baseline: 1.6492x; 1.6492x over previous
import jax
import jax.numpy as jnp
from jax import lax
from jax.experimental import pallas as pl
from jax.experimental.pallas import tpu as pltpu
from jax.experimental.pallas import tpu_sc as plsc

D_OUT = 4096
D_IN = 4096
BLOCK = 64
N_BLOCKS = D_IN // BLOCK
LANES = 16
GROUPS = N_BLOCKS // LANES

NC = 2
NS = 16
NW = NC * NS
ROWS_PER_W = D_OUT // NW
CHUNK_ROWS = 16
N_CHUNKS = ROWS_PER_W // CHUNK_ROWS


def _sc_body(master_hbm, scale_hbm, out_hbm, buf, sband):
    wid = lax.axis_index("s") * NC + lax.axis_index("c")
    base = wid * ROWS_PER_W

    pltpu.sync_copy(scale_hbm.at[pl.ds(base, ROWS_PER_W)], sband)

    def chunk_body(c, carry):
        r0 = base + c * CHUNK_ROWS
        pltpu.sync_copy(master_hbm.at[pl.ds(r0, CHUNK_ROWS)], buf)

        def row_group(i, carry2):
            r = i // GROUPS
            g = i % GROUPS
            sv = sband[c * CHUNK_ROWS + r, pl.ds(g * LANES, LANES)]
            ssafe = jnp.where(sv == 0.0, 1.0, sv)
            r75v = 7.5 / ssafe
            mv = sv * (2.0 / 15.0)
            dnums = lax.GatherDimensionNumbers(
                offset_dims=(), collapsed_slice_dims=(0,), start_index_map=(0,))
            for k in range(LANES):
                idx = jnp.full((LANES, 1), k, jnp.int32)
                r75s = lax.gather(r75v, idx, dnums, (1,),
                                  mode=lax.GatherScatterMode.PROMISE_IN_BOUNDS)
                ms = lax.gather(mv, idx, dnums, (1,),
                                mode=lax.GatherScatterMode.PROMISE_IN_BOUNDS)
                colbase = g * (LANES * BLOCK) + k * BLOCK
                for v in range(BLOCK // LANES):
                    col = colbase + v * LANES
                    x = buf[r, pl.ds(col, LANES)]
                    u = x * r75s + 8.0
                    cnt = u.astype(jnp.int32).astype(jnp.float32)
                    buf[r, pl.ds(col, LANES)] = (cnt - 7.5) * ms
            return carry2

        lax.fori_loop(0, CHUNK_ROWS * GROUPS, row_group, 0)
        pltpu.sync_copy(buf, out_hbm.at[pl.ds(r0, CHUNK_ROWS)])
        return carry

    lax.fori_loop(0, N_CHUNKS, chunk_body, 0)


def kernel(master, scale, centroids):
    del centroids
    mesh = plsc.VectorSubcoreMesh(core_axis_name="c", subcore_axis_name="s")
    k = pl.kernel(
        _sc_body,
        mesh=mesh,
        out_type=jax.ShapeDtypeStruct((D_OUT, D_IN), jnp.float32),
        scratch_types=[
            pltpu.VMEM((CHUNK_ROWS, D_IN), jnp.float32),
            pltpu.VMEM((ROWS_PER_W, N_BLOCKS), jnp.float32),
        ],
    )
    return k(master, scale)


# speedup vs baseline: 2.4883x; 1.5088x over previous
import jax
import jax.numpy as jnp
from jax import lax
from jax.experimental import pallas as pl
from jax.experimental.pallas import tpu as pltpu
from jax.experimental.pallas import tpu_sc as plsc

D_OUT = 4096
D_IN = 4096
BLOCK = 64
N_BLOCKS = D_IN // BLOCK
LANES = 16
GROUPS = N_BLOCKS // LANES

NC = 2
NS = 16
NW = NC * NS
ROWS_PER_W = D_OUT // NW
CHUNK_ROWS = 4
N_CHUNKS = ROWS_PER_W // CHUNK_ROWS
NBUF = 4
N_OUTER = N_CHUNKS // NBUF


def _compute_chunk(buf, sband, c):
    def row_group(i, carry):
        r = i // GROUPS
        g = i % GROUPS
        sv = sband[c * CHUNK_ROWS + r, pl.ds(g * LANES, LANES)]
        ssafe = jnp.where(sv == 0.0, 1.0, sv)
        r75v = 7.5 / ssafe
        mv = sv * (2.0 / 15.0)
        dnums = lax.GatherDimensionNumbers(
            offset_dims=(), collapsed_slice_dims=(0,), start_index_map=(0,))
        for k in range(LANES):
            idx = jnp.full((LANES, 1), k, jnp.int32)
            r75s = lax.gather(r75v, idx, dnums, (1,),
                              mode=lax.GatherScatterMode.PROMISE_IN_BOUNDS)
            ms = lax.gather(mv, idx, dnums, (1,),
                            mode=lax.GatherScatterMode.PROMISE_IN_BOUNDS)
            colbase = g * (LANES * BLOCK) + k * BLOCK
            for v in range(BLOCK // LANES):
                col = colbase + v * LANES
                x = buf[r, pl.ds(col, LANES)]
                u = x * r75s + 8.0
                cnt = u.astype(jnp.int32).astype(jnp.float32)
                buf[r, pl.ds(col, LANES)] = (cnt - 7.5) * ms
        return carry

    lax.fori_loop(0, CHUNK_ROWS * GROUPS, row_group, 0)


def _sc_body(master_hbm, scale_hbm, out_hbm,
             b0, b1, b2, b3, sband,
             is0, is1, is2, is3, os0, os1, os2, os3):
    bufs = (b0, b1, b2, b3)
    isems = (is0, is1, is2, is3)
    osems = (os0, os1, os2, os3)

    wid = lax.axis_index("s") * NC + lax.axis_index("c")
    base = wid * ROWS_PER_W

    pltpu.sync_copy(scale_hbm.at[pl.ds(base, ROWS_PER_W)], sband)

    def prefetch(c, j):
        pltpu.async_copy(master_hbm.at[pl.ds(base + c * CHUNK_ROWS, CHUNK_ROWS)],
                         bufs[j], isems[j])

    def wait_in(j):
        pltpu.make_async_copy(master_hbm.at[pl.ds(0, CHUNK_ROWS)],
                              bufs[j], isems[j]).wait()

    def wait_out(j):
        pltpu.make_async_copy(bufs[j], out_hbm.at[pl.ds(0, CHUNK_ROWS)],
                              osems[j]).wait()

    prefetch(0, 0)
    prefetch(1, 1)

    def outer_body(p, carry):
        for b in range(NBUF):
            c = p * NBUF + b
            wait_in(b)
            _compute_chunk(bufs[b], sband, c)
            pltpu.async_copy(bufs[b],
                             out_hbm.at[pl.ds(base + c * CHUNK_ROWS,
                                              CHUNK_ROWS)],
                             osems[b])
            nb = (b + 2) % NBUF

            @pl.when(c >= 2)
            def _():
                wait_out(nb)

            @pl.when(c + 2 < N_CHUNKS)
            def _():
                prefetch(c + 2, nb)
        return carry

    lax.fori_loop(0, N_OUTER, outer_body, 0)

    wait_out((N_CHUNKS - 2) % NBUF)
    wait_out((N_CHUNKS - 1) % NBUF)


def kernel(master, scale, centroids):
    del centroids
    mesh = plsc.VectorSubcoreMesh(core_axis_name="c", subcore_axis_name="s")
    k = pl.kernel(
        _sc_body,
        mesh=mesh,
        out_type=jax.ShapeDtypeStruct((D_OUT, D_IN), jnp.float32),
        scratch_types=[
            pltpu.VMEM((CHUNK_ROWS, D_IN), jnp.float32),
            pltpu.VMEM((CHUNK_ROWS, D_IN), jnp.float32),
            pltpu.VMEM((CHUNK_ROWS, D_IN), jnp.float32),
            pltpu.VMEM((CHUNK_ROWS, D_IN), jnp.float32),
            pltpu.VMEM((ROWS_PER_W, N_BLOCKS), jnp.float32),
            pltpu.SemaphoreType.DMA,
            pltpu.SemaphoreType.DMA,
            pltpu.SemaphoreType.DMA,
            pltpu.SemaphoreType.DMA,
            pltpu.SemaphoreType.DMA,
            pltpu.SemaphoreType.DMA,
            pltpu.SemaphoreType.DMA,
            pltpu.SemaphoreType.DMA,
        ],
    )
    return k(master, scale)
